# Initial kernel scaffold; baseline (speedup 1.0000x reference)
import jax
import jax.numpy as jnp
from jax.experimental import pallas as pl


def kernel(x, edge_index, W1, b1, gamma, beta, Wmu, bmu, Wls, bls):
    raise NotImplementedError("write your pallas kernel here")


# speedup vs baseline: 10.1795x; 10.1795x over previous
import jax
import jax.numpy as jnp
from jax import lax
from jax.experimental import pallas as pl
from jax.experimental.pallas import tpu as pltpu
from jax.experimental.pallas import tpu_sc as plsc

N = 10000
E = 320000
D = 128
H = D // 2
EPS = 1e-5

NC = 2
NS = 16
NW = NC * NS
EPT = E // NW
B = 80
STEPS = EPT // B
NPAD = 10240
RZ = NPAD // NW
RC = NPAD // NS

_MESH = plsc.VectorSubcoreMesh(core_axis_name="c", subcore_axis_name="s")


def _zero_fill(buf, rows, width):
    z = jnp.zeros((16,), jnp.float32)

    def row(i, carry):
        for j in range(width // 16):
            buf[i, pl.ds(j * 16, 16)] = z
        return carry

    lax.fori_loop(0, rows, row, 0)


def _sc_deg_body(dst_hbm, out_hbm, dst_v, ones_v, zbuf, acc):
    c = lax.axis_index("c")
    s = lax.axis_index("s")
    wid = s * NC + c

    one = jnp.ones((16,), jnp.float32)

    def orow(i, carry):
        ones_v[i, pl.ds(0, 16)] = one
        return carry

    lax.fori_loop(0, B, orow, 0)
    _zero_fill(zbuf, RZ, 16)
    pltpu.sync_copy(zbuf, acc.at[pl.ds(s * RC, RZ)])
    pltpu.sync_copy(zbuf, acc.at[pl.ds(s * RC + RZ, RZ)])
    plsc.subcore_barrier()

    ebase = wid * EPT

    def step(i, carry):
        off = pl.multiple_of(ebase + i * B, 8)
        pltpu.sync_copy(dst_hbm.at[pl.ds(off, B)], dst_v)
        pltpu.sync_copy(ones_v, acc.at[dst_v], add=True)
        return carry

    lax.fori_loop(0, STEPS, step, 0)
    plsc.subcore_barrier()
    pltpu.sync_copy(acc.at[pl.ds(s * RC, RC)], out_hbm.at[c, pl.ds(s * RC, RC)])


_sc_deg = pl.kernel(
    _sc_deg_body,
    out_type=jax.ShapeDtypeStruct((NC, NPAD, 16), jnp.float32),
    mesh=_MESH,
    compiler_params=pltpu.CompilerParams(use_tc_tiling_on_sc=False),
    scratch_types=[
        pltpu.VMEM((B,), jnp.int32),
        pltpu.VMEM((B, 16), jnp.float32),
        pltpu.VMEM((RZ, 16), jnp.float32),
        pltpu.VMEM_SHARED((NPAD, 16), jnp.float32),
    ],
)


def _sc_agg_body(ulo_hbm, uhi_hbm, src_hbm, dst_hbm, out_hbm, src_v, dst_v,
                 rows_v, zbuf, acc, sem):
    c = lax.axis_index("c")
    s = lax.axis_index("s")
    wid = s * NC + c
    ebase = wid * EPT

    _zero_fill(zbuf, RZ, H)

    for half, u_hbm in ((0, ulo_hbm), (1, uhi_hbm)):
        pltpu.sync_copy(zbuf, acc.at[pl.ds(s * RC, RZ)])
        pltpu.sync_copy(zbuf, acc.at[pl.ds(s * RC + RZ, RZ)])
        plsc.subcore_barrier()

        def step(i, carry):
            off = pl.multiple_of(ebase + i * B, 8)
            pltpu.sync_copy(src_hbm.at[pl.ds(off, B)], src_v)
            pltpu.sync_copy(dst_hbm.at[pl.ds(off, B)], dst_v)
            pltpu.async_copy(u_hbm.at[src_v], rows_v, sem).wait()
            pltpu.sync_copy(rows_v, acc.at[dst_v], add=True)
            return carry

        lax.fori_loop(0, STEPS, step, 0)
        plsc.subcore_barrier()
        pltpu.sync_copy(acc.at[pl.ds(s * RC, RC)],
                        out_hbm.at[c, half, pl.ds(s * RC, RC)])
        plsc.subcore_barrier()


_sc_agg = pl.kernel(
    _sc_agg_body,
    out_type=jax.ShapeDtypeStruct((NC, 2, NPAD, H), jnp.float32),
    mesh=_MESH,
    compiler_params=pltpu.CompilerParams(use_tc_tiling_on_sc=False),
    scratch_types=[
        pltpu.VMEM((B,), jnp.int32),
        pltpu.VMEM((B,), jnp.int32),
        pltpu.VMEM((B, H), jnp.float32),
        pltpu.VMEM((RZ, H), jnp.float32),
        pltpu.VMEM_SHARED((NPAD, H), jnp.float32),
        pltpu.SemaphoreType.DMA,
    ],
)


def _tc_a_body(d0_ref, d1_ref, x_ref, ulo_ref, uhi_ref, dinv_ref):
    deg = d0_ref[:, 0:1] + d1_ref[:, 0:1] + 1.0
    dinv = lax.rsqrt(deg)
    dinv_ref[...] = dinv
    u = x_ref[...] * dinv
    ulo_ref[...] = u[:, :H]
    uhi_ref[...] = u[:, H:]


_tc_a = pl.pallas_call(
    _tc_a_body,
    out_shape=(
        jax.ShapeDtypeStruct((N, H), jnp.float32),
        jax.ShapeDtypeStruct((N, H), jnp.float32),
        jax.ShapeDtypeStruct((N, 1), jnp.float32),
    ),
)


def _assemble(p_ref, ulo_ref, uhi_ref, dinv):
    lo = p_ref[0, 0, :N, :] + p_ref[1, 0, :N, :] + ulo_ref[...]
    hi = p_ref[0, 1, :N, :] + p_ref[1, 1, :N, :] + uhi_ref[...]
    return jnp.concatenate([lo, hi], axis=1) * dinv


def _tc_b_body(p_ref, ulo_ref, uhi_ref, dinv_ref, W1_ref, g_ref, b_ref,
               u2lo_ref, u2hi_ref):
    dinv = dinv_ref[...]
    agg = _assemble(p_ref, ulo_ref, uhi_ref, dinv)
    hp = jnp.dot(agg, W1_ref[...], preferred_element_type=jnp.float32)
    m = jnp.mean(hp, axis=0, keepdims=True)
    v = jnp.mean((hp - m) ** 2, axis=0, keepdims=True)
    h = (hp - m) * lax.rsqrt(v + EPS) * g_ref[...] + b_ref[...]
    h = jnp.maximum(h, 0.0)
    u2 = h * dinv
    u2lo_ref[...] = u2[:, :H]
    u2hi_ref[...] = u2[:, H:]


_tc_b = pl.pallas_call(
    _tc_b_body,
    out_shape=(
        jax.ShapeDtypeStruct((N, H), jnp.float32),
        jax.ShapeDtypeStruct((N, H), jnp.float32),
    ),
)


def _tc_c_body(p_ref, u2lo_ref, u2hi_ref, dinv_ref, Wmu_ref, bmu_ref,
               Wls_ref, bls_ref, mu_ref, ls_ref):
    agg = _assemble(p_ref, u2lo_ref, u2hi_ref, dinv_ref[...])
    mu_ref[...] = jnp.dot(agg, Wmu_ref[...],
                          preferred_element_type=jnp.float32) + bmu_ref[...]
    ls_ref[...] = jnp.dot(agg, Wls_ref[...],
                          preferred_element_type=jnp.float32) + bls_ref[...]


_tc_c = pl.pallas_call(
    _tc_c_body,
    out_shape=(
        jax.ShapeDtypeStruct((N, D), jnp.float32),
        jax.ShapeDtypeStruct((N, D), jnp.float32),
    ),
)


@jax.jit
def kernel(x, edge_index, W1, b1, gamma, beta, Wmu, bmu, Wls, bls):
    src = edge_index[0]
    dst = edge_index[1]

    degp = _sc_deg(dst)
    ulo, uhi, dinv = _tc_a(degp[0, :N], degp[1, :N], x)

    p1 = _sc_agg(ulo, uhi, src, dst)
    u2lo, u2hi = _tc_b(p1, ulo, uhi, dinv, W1, gamma, beta)

    p2 = _sc_agg(u2lo, u2hi, src, dst)
    mu, ls = _tc_c(p2, u2lo, u2hi, dinv, Wmu, bmu, Wls, bls)
    return (mu, ls)


# speedup vs baseline: 29.4935x; 2.8973x over previous
import jax
import jax.numpy as jnp
from jax import lax
from jax.experimental import pallas as pl
from jax.experimental.pallas import tpu as pltpu
from jax.experimental.pallas import tpu_sc as plsc

N = 10000
E = 320000
D = 128
H = D // 2
EPS = 1e-5

NC = 2
NS = 16
NW = NC * NS
EPT = E // NW
B = 100
STEPS = EPT // B
NBUF = 4
KDEG = 10
NPAD = 10240
RZ = NPAD // NW
RC = NPAD // NS

_MESH = plsc.VectorSubcoreMesh(core_axis_name="c", subcore_axis_name="s")


def _zero_fill(buf, rows, width):
    z = jnp.zeros((16,), jnp.float32)

    def row(i, carry):
        for j in range(width // 16):
            buf[i, pl.ds(j * 16, 16)] = z
        return carry

    lax.fori_loop(0, rows, row, 0)


def _sc_deg_body(dstr_hbm, out_hbm, dst_t, ones_v, zbuf, acc, *sems):
    c = lax.axis_index("c")
    s = lax.axis_index("s")
    wid = s * NC + c

    one = jnp.ones((16,), jnp.float32)

    def orow(i, carry):
        ones_v[i, pl.ds(0, 16)] = one
        return carry

    lax.fori_loop(0, B, orow, 0)
    pltpu.sync_copy(dstr_hbm.at[wid], dst_t)
    _zero_fill(zbuf, RZ, 16)
    pltpu.sync_copy(zbuf, acc.at[pl.ds(s * RC, RZ)])
    pltpu.sync_copy(zbuf, acc.at[pl.ds(s * RC + RZ, RZ)])
    plsc.subcore_barrier()

    def group(g, carry):
        for b in range(KDEG):
            j = g * KDEG + b

            @pl.when(j >= KDEG)
            def _():
                pltpu.make_async_copy(ones_v, acc.at[dst_t.at[0]],
                                      sems[b]).wait()

            pltpu.make_async_copy(ones_v, acc.at[dst_t.at[j]],
                                  sems[b]).start(add=True)
        return carry

    lax.fori_loop(0, STEPS // KDEG, group, 0)
    for b in range(KDEG):
        pltpu.make_async_copy(ones_v, acc.at[dst_t.at[0]], sems[b]).wait()
    plsc.subcore_barrier()
    pltpu.sync_copy(acc.at[pl.ds(s * RC, RC)], out_hbm.at[c, pl.ds(s * RC, RC)])


_sc_deg = pl.kernel(
    _sc_deg_body,
    out_type=jax.ShapeDtypeStruct((NC, NPAD, 16), jnp.float32),
    mesh=_MESH,
    compiler_params=pltpu.CompilerParams(use_tc_tiling_on_sc=False),
    scratch_types=[
        pltpu.VMEM((STEPS, B), jnp.int32),
        pltpu.VMEM((B, 16), jnp.float32),
        pltpu.VMEM((RZ, 16), jnp.float32),
        pltpu.VMEM_SHARED((NPAD, 16), jnp.float32),
    ] + [pltpu.SemaphoreType.DMA] * KDEG,
)


def _sc_agg_body(ulo_hbm, uhi_hbm, srcr_hbm, dstr_hbm, out_hbm, src_t, dst_t,
                 r0, r1, r2, r3, zbuf, acc, *sems):
    c = lax.axis_index("c")
    s = lax.axis_index("s")
    wid = s * NC + c
    rows = (r0, r1, r2, r3)
    gsem = sems[:NBUF]
    ssem = sems[NBUF:]

    pltpu.sync_copy(srcr_hbm.at[wid], src_t)
    pltpu.sync_copy(dstr_hbm.at[wid], dst_t)
    _zero_fill(zbuf, RZ, H)

    for half, u_hbm in ((0, ulo_hbm), (1, uhi_hbm)):
        pltpu.sync_copy(zbuf, acc.at[pl.ds(s * RC, RZ)])
        pltpu.sync_copy(zbuf, acc.at[pl.ds(s * RC + RZ, RZ)])
        plsc.subcore_barrier()

        for b in range(2):
            pltpu.make_async_copy(u_hbm.at[src_t.at[b]], rows[b],
                                  gsem[b]).start()

        def group(g, carry):
            for b in range(NBUF):
                j = g * NBUF + b
                pltpu.make_async_copy(u_hbm.at[src_t.at[0]], rows[b],
                                      gsem[b]).wait()
                pltpu.make_async_copy(rows[b], acc.at[dst_t.at[j]],
                                      ssem[b]).start(add=True)
                nj = j + 2
                b2 = (b + 2) % NBUF

                @pl.when(nj < STEPS)
                def _():
                    @pl.when(j >= 2)
                    def _():
                        pltpu.make_async_copy(rows[b2], acc.at[dst_t.at[0]],
                                              ssem[b2]).wait()

                    pltpu.make_async_copy(u_hbm.at[src_t.at[nj]], rows[b2],
                                          gsem[b2]).start()
            return carry

        lax.fori_loop(0, STEPS // NBUF, group, 0)
        for b in range(NBUF):
            pltpu.make_async_copy(rows[b], acc.at[dst_t.at[0]],
                                  ssem[b]).wait()
        plsc.subcore_barrier()
        pltpu.sync_copy(acc.at[pl.ds(s * RC, RC)],
                        out_hbm.at[c, half, pl.ds(s * RC, RC)])
        plsc.subcore_barrier()


_sc_agg = pl.kernel(
    _sc_agg_body,
    out_type=jax.ShapeDtypeStruct((NC, 2, NPAD, H), jnp.float32),
    mesh=_MESH,
    compiler_params=pltpu.CompilerParams(use_tc_tiling_on_sc=False),
    scratch_types=[
        pltpu.VMEM((STEPS, B), jnp.int32),
        pltpu.VMEM((STEPS, B), jnp.int32),
    ] + [pltpu.VMEM((B, H), jnp.float32)] * NBUF + [
        pltpu.VMEM((RZ, H), jnp.float32),
        pltpu.VMEM_SHARED((NPAD, H), jnp.float32),
    ] + [pltpu.SemaphoreType.DMA] * (2 * NBUF),
)


def _tc_a_body(d0_ref, d1_ref, x_ref, ulo_ref, uhi_ref, dinv_ref):
    deg = d0_ref[:, 0:1] + d1_ref[:, 0:1] + 1.0
    dinv = lax.rsqrt(deg)
    dinv_ref[...] = dinv
    u = x_ref[...] * dinv
    ulo_ref[...] = u[:, :H]
    uhi_ref[...] = u[:, H:]


_tc_a = pl.pallas_call(
    _tc_a_body,
    out_shape=(
        jax.ShapeDtypeStruct((N, H), jnp.float32),
        jax.ShapeDtypeStruct((N, H), jnp.float32),
        jax.ShapeDtypeStruct((N, 1), jnp.float32),
    ),
)


def _assemble(p_ref, ulo_ref, uhi_ref, dinv):
    lo = p_ref[0, 0, :N, :] + p_ref[1, 0, :N, :] + ulo_ref[...]
    hi = p_ref[0, 1, :N, :] + p_ref[1, 1, :N, :] + uhi_ref[...]
    return jnp.concatenate([lo, hi], axis=1) * dinv


def _tc_b_body(p_ref, ulo_ref, uhi_ref, dinv_ref, W1_ref, g_ref, b_ref,
               u2lo_ref, u2hi_ref):
    dinv = dinv_ref[...]
    agg = _assemble(p_ref, ulo_ref, uhi_ref, dinv)
    hp = jnp.dot(agg, W1_ref[...], preferred_element_type=jnp.float32)
    m = jnp.mean(hp, axis=0, keepdims=True)
    v = jnp.mean((hp - m) ** 2, axis=0, keepdims=True)
    h = (hp - m) * lax.rsqrt(v + EPS) * g_ref[...] + b_ref[...]
    h = jnp.maximum(h, 0.0)
    u2 = h * dinv
    u2lo_ref[...] = u2[:, :H]
    u2hi_ref[...] = u2[:, H:]


_tc_b = pl.pallas_call(
    _tc_b_body,
    out_shape=(
        jax.ShapeDtypeStruct((N, H), jnp.float32),
        jax.ShapeDtypeStruct((N, H), jnp.float32),
    ),
)


def _tc_c_body(p_ref, u2lo_ref, u2hi_ref, dinv_ref, Wmu_ref, bmu_ref,
               Wls_ref, bls_ref, mu_ref, ls_ref):
    agg = _assemble(p_ref, u2lo_ref, u2hi_ref, dinv_ref[...])
    mu_ref[...] = jnp.dot(agg, Wmu_ref[...],
                          preferred_element_type=jnp.float32) + bmu_ref[...]
    ls_ref[...] = jnp.dot(agg, Wls_ref[...],
                          preferred_element_type=jnp.float32) + bls_ref[...]


_tc_c = pl.pallas_call(
    _tc_c_body,
    out_shape=(
        jax.ShapeDtypeStruct((N, D), jnp.float32),
        jax.ShapeDtypeStruct((N, D), jnp.float32),
    ),
)


@jax.jit
def kernel(x, edge_index, W1, b1, gamma, beta, Wmu, bmu, Wls, bls):
    src = edge_index[0].reshape(NW, STEPS, B)
    dst = edge_index[1].reshape(NW, STEPS, B)

    degp = _sc_deg(dst)
    ulo, uhi, dinv = _tc_a(degp[0, :N], degp[1, :N], x)

    p1 = _sc_agg(ulo, uhi, src, dst)
    u2lo, u2hi = _tc_b(p1, ulo, uhi, dinv, W1, gamma, beta)

    p2 = _sc_agg(u2lo, u2hi, src, dst)
    mu, ls = _tc_c(p2, u2lo, u2hi, dinv, Wmu, bmu, Wls, bls)
    return (mu, ls)


# speedup vs baseline: 30.3783x; 1.0300x over previous
import jax
import jax.numpy as jnp
from jax import lax
from jax.experimental import pallas as pl
from jax.experimental.pallas import tpu as pltpu
from jax.experimental.pallas import tpu_sc as plsc

N = 10000
E = 320000
D = 128
H = D // 2
EPS = 1e-5

NC = 2
NS = 16
NW = NC * NS
EPT = E // NW
B = 100
STEPS = EPT // B
NBUF = 4
KDEG = 10
NPAD = 10240
RZ = NPAD // NW
RC = NPAD // NS

_MESH = plsc.VectorSubcoreMesh(core_axis_name="c", subcore_axis_name="s")


def _zero_fill(buf, rows, width):
    z = jnp.zeros((16,), jnp.float32)

    def row(i, carry):
        for j in range(width // 16):
            buf[i, pl.ds(j * 16, 16)] = z
        return carry

    lax.fori_loop(0, rows, row, 0)


def _sc_deg_body(er_hbm, out_hbm, dst_t, ones_v, zbuf, acc, *sems):
    c = lax.axis_index("c")
    s = lax.axis_index("s")
    wid = s * NC + c

    one = jnp.ones((16,), jnp.float32)

    def orow(i, carry):
        ones_v[i, pl.ds(0, 16)] = one
        return carry

    lax.fori_loop(0, B, orow, 0)
    pltpu.sync_copy(er_hbm.at[1, wid], dst_t)
    _zero_fill(zbuf, RZ, 16)
    pltpu.sync_copy(zbuf, acc.at[pl.ds(s * RC, RZ)])
    pltpu.sync_copy(zbuf, acc.at[pl.ds(s * RC + RZ, RZ)])
    plsc.subcore_barrier()

    def group(g, carry):
        for b in range(KDEG):
            j = g * KDEG + b

            @pl.when(j >= KDEG)
            def _():
                pltpu.make_async_copy(ones_v, acc.at[dst_t.at[0]],
                                      sems[b]).wait()

            pltpu.make_async_copy(ones_v, acc.at[dst_t.at[j]],
                                  sems[b]).start(add=True)
        return carry

    lax.fori_loop(0, STEPS // KDEG, group, 0)
    for b in range(KDEG):
        pltpu.make_async_copy(ones_v, acc.at[dst_t.at[0]], sems[b]).wait()
    plsc.subcore_barrier()
    pltpu.sync_copy(acc.at[pl.ds(s * RC, RC)], out_hbm.at[c, pl.ds(s * RC, RC)])


_sc_deg = pl.kernel(
    _sc_deg_body,
    out_type=jax.ShapeDtypeStruct((NC, NPAD, 16), jnp.float32),
    mesh=_MESH,
    compiler_params=pltpu.CompilerParams(use_tc_tiling_on_sc=False),
    scratch_types=[
        pltpu.VMEM((STEPS, B), jnp.int32),
        pltpu.VMEM((B, 16), jnp.float32),
        pltpu.VMEM((RZ, 16), jnp.float32),
        pltpu.VMEM_SHARED((NPAD, 16), jnp.float32),
    ] + [pltpu.SemaphoreType.DMA] * KDEG,
)


def _sc_agg_body(ulo_hbm, uhi_hbm, er_hbm, out_hbm, src_t, dst_t,
                 r0, r1, r2, r3, zbuf, acc, *sems):
    c = lax.axis_index("c")
    s = lax.axis_index("s")
    wid = s * NC + c
    rows = (r0, r1, r2, r3)
    gsem = sems[:NBUF]
    ssem = sems[NBUF:]

    pltpu.sync_copy(er_hbm.at[0, wid], src_t)
    pltpu.sync_copy(er_hbm.at[1, wid], dst_t)
    _zero_fill(zbuf, RZ, H)

    for half, u_hbm in ((0, ulo_hbm), (1, uhi_hbm)):
        pltpu.sync_copy(zbuf, acc.at[pl.ds(s * RC, RZ)])
        pltpu.sync_copy(zbuf, acc.at[pl.ds(s * RC + RZ, RZ)])
        plsc.subcore_barrier()

        for b in range(2):
            pltpu.make_async_copy(u_hbm.at[src_t.at[b]], rows[b],
                                  gsem[b]).start()

        def group(g, carry):
            for b in range(NBUF):
                j = g * NBUF + b
                pltpu.make_async_copy(u_hbm.at[src_t.at[0]], rows[b],
                                      gsem[b]).wait()
                pltpu.make_async_copy(rows[b], acc.at[dst_t.at[j]],
                                      ssem[b]).start(add=True)
                nj = j + 2
                b2 = (b + 2) % NBUF

                @pl.when(nj < STEPS)
                def _():
                    @pl.when(j >= 2)
                    def _():
                        pltpu.make_async_copy(rows[b2], acc.at[dst_t.at[0]],
                                              ssem[b2]).wait()

                    pltpu.make_async_copy(u_hbm.at[src_t.at[nj]], rows[b2],
                                          gsem[b2]).start()
            return carry

        lax.fori_loop(0, STEPS // NBUF, group, 0)
        for b in range(NBUF):
            pltpu.make_async_copy(rows[b], acc.at[dst_t.at[0]],
                                  ssem[b]).wait()
        plsc.subcore_barrier()
        pltpu.sync_copy(acc.at[pl.ds(s * RC, RC)],
                        out_hbm.at[c, half, pl.ds(s * RC, RC)])
        if half == 0:
            plsc.subcore_barrier()


_sc_agg = pl.kernel(
    _sc_agg_body,
    out_type=jax.ShapeDtypeStruct((NC, 2, NPAD, H), jnp.float32),
    mesh=_MESH,
    compiler_params=pltpu.CompilerParams(use_tc_tiling_on_sc=False),
    scratch_types=[
        pltpu.VMEM((STEPS, B), jnp.int32),
        pltpu.VMEM((STEPS, B), jnp.int32),
    ] + [pltpu.VMEM((B, H), jnp.float32)] * NBUF + [
        pltpu.VMEM((RZ, H), jnp.float32),
        pltpu.VMEM_SHARED((NPAD, H), jnp.float32),
    ] + [pltpu.SemaphoreType.DMA] * (2 * NBUF),
)


def _tc_a_body(degp_ref, x_ref, ulo_ref, uhi_ref, dinv_ref):
    deg = (degp_ref[0, :N, 0:1] + degp_ref[1, :N, 0:1]
           + 1.0)
    dinv = lax.rsqrt(deg)
    dinv_ref[...] = dinv
    u = x_ref[...] * dinv
    ulo_ref[...] = u[:, :H]
    uhi_ref[...] = u[:, H:]


_tc_a = pl.pallas_call(
    _tc_a_body,
    out_shape=(
        jax.ShapeDtypeStruct((N, H), jnp.float32),
        jax.ShapeDtypeStruct((N, H), jnp.float32),
        jax.ShapeDtypeStruct((N, 1), jnp.float32),
    ),
)


def _assemble(p_ref, ulo_ref, uhi_ref, dinv):
    lo = p_ref[0, 0, :N, :] + p_ref[1, 0, :N, :] + ulo_ref[...]
    hi = p_ref[0, 1, :N, :] + p_ref[1, 1, :N, :] + uhi_ref[...]
    return jnp.concatenate([lo, hi], axis=1) * dinv


def _tc_b_body(p_ref, ulo_ref, uhi_ref, dinv_ref, W1_ref, g_ref, b_ref,
               u2lo_ref, u2hi_ref):
    dinv = dinv_ref[...]
    agg = _assemble(p_ref, ulo_ref, uhi_ref, dinv)
    hp = jnp.dot(agg, W1_ref[...], preferred_element_type=jnp.float32)
    m = jnp.mean(hp, axis=0, keepdims=True)
    v = jnp.mean((hp - m) ** 2, axis=0, keepdims=True)
    h = (hp - m) * lax.rsqrt(v + EPS) * g_ref[...] + b_ref[...]
    h = jnp.maximum(h, 0.0)
    u2 = h * dinv
    u2lo_ref[...] = u2[:, :H]
    u2hi_ref[...] = u2[:, H:]


_tc_b = pl.pallas_call(
    _tc_b_body,
    out_shape=(
        jax.ShapeDtypeStruct((N, H), jnp.float32),
        jax.ShapeDtypeStruct((N, H), jnp.float32),
    ),
)


def _tc_c_body(p_ref, u2lo_ref, u2hi_ref, dinv_ref, Wmu_ref, bmu_ref,
               Wls_ref, bls_ref, mu_ref, ls_ref):
    agg = _assemble(p_ref, u2lo_ref, u2hi_ref, dinv_ref[...])
    mu_ref[...] = jnp.dot(agg, Wmu_ref[...],
                          preferred_element_type=jnp.float32) + bmu_ref[...]
    ls_ref[...] = jnp.dot(agg, Wls_ref[...],
                          preferred_element_type=jnp.float32) + bls_ref[...]


_tc_c = pl.pallas_call(
    _tc_c_body,
    out_shape=(
        jax.ShapeDtypeStruct((N, D), jnp.float32),
        jax.ShapeDtypeStruct((N, D), jnp.float32),
    ),
)


@jax.jit
def kernel(x, edge_index, W1, b1, gamma, beta, Wmu, bmu, Wls, bls):
    er = edge_index.reshape(2, NW, STEPS, B)

    degp = _sc_deg(er)
    ulo, uhi, dinv = _tc_a(degp, x)

    p1 = _sc_agg(ulo, uhi, er)
    u2lo, u2hi = _tc_b(p1, ulo, uhi, dinv, W1, gamma, beta)

    p2 = _sc_agg(u2lo, u2hi, er)
    mu, ls = _tc_c(p2, u2lo, u2hi, dinv, Wmu, bmu, Wls, bls)
    return (mu, ls)


# speedup vs baseline: 31.7307x; 1.0445x over previous
import jax
import jax.numpy as jnp
from jax import lax
from jax.experimental import pallas as pl
from jax.experimental.pallas import tpu as pltpu
from jax.experimental.pallas import tpu_sc as plsc

N = 10000
E = 320000
D = 128
H = D // 2
EPS = 1e-5

NC = 2
NS = 16
NW = NC * NS
EPT = E // NW
B = 125
STEPS = EPT // B
NBUF = 4
KDEG = 10
NPAD = 10240
RZ = NPAD // NW
RC = NPAD // NS

_MESH = plsc.VectorSubcoreMesh(core_axis_name="c", subcore_axis_name="s")


def _zero_fill(buf, rows, width):
    z = jnp.zeros((16,), jnp.float32)

    def row(i, carry):
        for j in range(width // 16):
            buf[i, pl.ds(j * 16, 16)] = z
        return carry

    lax.fori_loop(0, rows, row, 0)


def _sc_deg_body(er_hbm, out_hbm, dst_t, ones_v, zbuf, acc, *sems):
    c = lax.axis_index("c")
    s = lax.axis_index("s")
    wid = s * NC + c

    one = jnp.ones((16,), jnp.float32)

    def orow(i, carry):
        ones_v[i, pl.ds(0, 16)] = one
        return carry

    lax.fori_loop(0, B, orow, 0)
    pltpu.sync_copy(er_hbm.at[1, wid], dst_t)
    _zero_fill(zbuf, RZ, 16)
    pltpu.sync_copy(zbuf, acc.at[pl.ds(s * RC, RZ)])
    pltpu.sync_copy(zbuf, acc.at[pl.ds(s * RC + RZ, RZ)])
    plsc.subcore_barrier()

    def group(g, carry):
        for b in range(KDEG):
            j = g * KDEG + b

            @pl.when(j >= KDEG)
            def _():
                pltpu.make_async_copy(ones_v, acc.at[dst_t.at[0]],
                                      sems[b]).wait()

            pltpu.make_async_copy(ones_v, acc.at[dst_t.at[j]],
                                  sems[b]).start(add=True)
        return carry

    lax.fori_loop(0, STEPS // KDEG, group, 0)
    for b in range(KDEG):
        pltpu.make_async_copy(ones_v, acc.at[dst_t.at[0]], sems[b]).wait()
    plsc.subcore_barrier()
    pltpu.sync_copy(acc.at[pl.ds(s * RC, RC)], out_hbm.at[c, pl.ds(s * RC, RC)])


_sc_deg = pl.kernel(
    _sc_deg_body,
    out_type=jax.ShapeDtypeStruct((NC, NPAD, 16), jnp.float32),
    mesh=_MESH,
    compiler_params=pltpu.CompilerParams(use_tc_tiling_on_sc=False),
    scratch_types=[
        pltpu.VMEM((STEPS, B), jnp.int32),
        pltpu.VMEM((B, 16), jnp.float32),
        pltpu.VMEM((RZ, 16), jnp.float32),
        pltpu.VMEM_SHARED((NPAD, 16), jnp.float32),
    ] + [pltpu.SemaphoreType.DMA] * KDEG,
)


def _sc_agg_body(ulo_hbm, uhi_hbm, er_hbm, out_hbm, src_t, dst_t,
                 r0, r1, r2, r3, zbuf, acc, *sems):
    c = lax.axis_index("c")
    s = lax.axis_index("s")
    wid = s * NC + c
    rows = (r0, r1, r2, r3)
    gsem = sems[:NBUF]
    ssem = sems[NBUF:]

    pltpu.sync_copy(er_hbm.at[0, wid], src_t)
    pltpu.sync_copy(er_hbm.at[1, wid], dst_t)
    _zero_fill(zbuf, RZ, H)

    for half, u_hbm in ((0, ulo_hbm), (1, uhi_hbm)):
        pltpu.sync_copy(zbuf, acc.at[pl.ds(s * RC, RZ)])
        pltpu.sync_copy(zbuf, acc.at[pl.ds(s * RC + RZ, RZ)])
        plsc.subcore_barrier()

        for b in range(2):
            pltpu.make_async_copy(u_hbm.at[src_t.at[b]], rows[b],
                                  gsem[b]).start()

        def group(g, carry):
            for b in range(NBUF):
                j = g * NBUF + b
                pltpu.make_async_copy(u_hbm.at[src_t.at[0]], rows[b],
                                      gsem[b]).wait()
                pltpu.make_async_copy(rows[b], acc.at[dst_t.at[j]],
                                      ssem[b]).start(add=True)
                nj = j + 2
                b2 = (b + 2) % NBUF

                @pl.when(nj < STEPS)
                def _():
                    @pl.when(j >= 2)
                    def _():
                        pltpu.make_async_copy(rows[b2], acc.at[dst_t.at[0]],
                                              ssem[b2]).wait()

                    pltpu.make_async_copy(u_hbm.at[src_t.at[nj]], rows[b2],
                                          gsem[b2]).start()
            return carry

        lax.fori_loop(0, STEPS // NBUF, group, 0)
        for b in range(NBUF):
            pltpu.make_async_copy(rows[b], acc.at[dst_t.at[0]],
                                  ssem[b]).wait()
        plsc.subcore_barrier()
        pltpu.sync_copy(acc.at[pl.ds(s * RC, RC)],
                        out_hbm.at[c, half, pl.ds(s * RC, RC)])
        if half == 0:
            plsc.subcore_barrier()


_sc_agg = pl.kernel(
    _sc_agg_body,
    out_type=jax.ShapeDtypeStruct((NC, 2, NPAD, H), jnp.float32),
    mesh=_MESH,
    compiler_params=pltpu.CompilerParams(use_tc_tiling_on_sc=False),
    scratch_types=[
        pltpu.VMEM((STEPS, B), jnp.int32),
        pltpu.VMEM((STEPS, B), jnp.int32),
    ] + [pltpu.VMEM((B, H), jnp.float32)] * NBUF + [
        pltpu.VMEM((RZ, H), jnp.float32),
        pltpu.VMEM_SHARED((NPAD, H), jnp.float32),
    ] + [pltpu.SemaphoreType.DMA] * (2 * NBUF),
)


def _tc_a_body(degp_ref, x_ref, ulo_ref, uhi_ref, dinv_ref):
    deg = (degp_ref[0, :N, 0:1] + degp_ref[1, :N, 0:1]
           + 1.0)
    dinv = lax.rsqrt(deg)
    dinv_ref[...] = dinv
    u = x_ref[...] * dinv
    ulo_ref[...] = u[:, :H]
    uhi_ref[...] = u[:, H:]


_tc_a = pl.pallas_call(
    _tc_a_body,
    out_shape=(
        jax.ShapeDtypeStruct((N, H), jnp.float32),
        jax.ShapeDtypeStruct((N, H), jnp.float32),
        jax.ShapeDtypeStruct((N, 1), jnp.float32),
    ),
)


def _assemble(p_ref, ulo_ref, uhi_ref, dinv):
    lo = p_ref[0, 0, :N, :] + p_ref[1, 0, :N, :] + ulo_ref[...]
    hi = p_ref[0, 1, :N, :] + p_ref[1, 1, :N, :] + uhi_ref[...]
    return jnp.concatenate([lo, hi], axis=1) * dinv


def _tc_b_body(p_ref, ulo_ref, uhi_ref, dinv_ref, W1_ref, g_ref, b_ref,
               u2lo_ref, u2hi_ref):
    dinv = dinv_ref[...]
    agg = _assemble(p_ref, ulo_ref, uhi_ref, dinv)
    hp = jnp.dot(agg, W1_ref[...], preferred_element_type=jnp.float32)
    m = jnp.mean(hp, axis=0, keepdims=True)
    v = jnp.mean((hp - m) ** 2, axis=0, keepdims=True)
    h = (hp - m) * lax.rsqrt(v + EPS) * g_ref[...] + b_ref[...]
    h = jnp.maximum(h, 0.0)
    u2 = h * dinv
    u2lo_ref[...] = u2[:, :H]
    u2hi_ref[...] = u2[:, H:]


_tc_b = pl.pallas_call(
    _tc_b_body,
    out_shape=(
        jax.ShapeDtypeStruct((N, H), jnp.float32),
        jax.ShapeDtypeStruct((N, H), jnp.float32),
    ),
)


def _tc_c_body(p_ref, u2lo_ref, u2hi_ref, dinv_ref, Wmu_ref, bmu_ref,
               Wls_ref, bls_ref, mu_ref, ls_ref):
    agg = _assemble(p_ref, u2lo_ref, u2hi_ref, dinv_ref[...])
    mu_ref[...] = jnp.dot(agg, Wmu_ref[...],
                          preferred_element_type=jnp.float32) + bmu_ref[...]
    ls_ref[...] = jnp.dot(agg, Wls_ref[...],
                          preferred_element_type=jnp.float32) + bls_ref[...]


_tc_c = pl.pallas_call(
    _tc_c_body,
    out_shape=(
        jax.ShapeDtypeStruct((N, D), jnp.float32),
        jax.ShapeDtypeStruct((N, D), jnp.float32),
    ),
)


@jax.jit
def kernel(x, edge_index, W1, b1, gamma, beta, Wmu, bmu, Wls, bls):
    er = edge_index.reshape(2, NW, STEPS, B)

    degp = _sc_deg(er)
    ulo, uhi, dinv = _tc_a(degp, x)

    p1 = _sc_agg(ulo, uhi, er)
    u2lo, u2hi = _tc_b(p1, ulo, uhi, dinv, W1, gamma, beta)

    p2 = _sc_agg(u2lo, u2hi, er)
    mu, ls = _tc_c(p2, u2lo, u2hi, dinv, Wmu, bmu, Wls, bls)
    return (mu, ls)


# speedup vs baseline: 31.9904x; 1.0082x over previous
import jax
import jax.numpy as jnp
from jax import lax
from jax.experimental import pallas as pl
from jax.experimental.pallas import tpu as pltpu
from jax.experimental.pallas import tpu_sc as plsc

N = 10000
E = 320000
D = 128
H = D // 2
EPS = 1e-5

NC = 2
NS = 16
NW = NC * NS
EPT = E // NW
B = 125
STEPS = EPT // B
NBUF = 4
KDEG = 10
NPAD = 10240
RZ = NPAD // NW
RC = NPAD // NS

_MESH = plsc.VectorSubcoreMesh(core_axis_name="c", subcore_axis_name="s")


def _zero_fill(buf, rows, width):
    z = jnp.zeros((16,), jnp.float32)

    def row(i, carry):
        for j in range(width // 16):
            buf[i, pl.ds(j * 16, 16)] = z
        return carry

    lax.fori_loop(0, rows, row, 0)


def _sc_deg_body(er_hbm, out_hbm, dst_t, ones_v, zbuf, acc, *sems):
    c = lax.axis_index("c")
    s = lax.axis_index("s")
    wid = s * NC + c

    one = jnp.ones((16,), jnp.float32)

    def orow(i, carry):
        ones_v[i, pl.ds(0, 16)] = one
        return carry

    lax.fori_loop(0, B, orow, 0)
    pltpu.sync_copy(er_hbm.at[1, wid], dst_t)
    _zero_fill(zbuf, RZ, 16)
    pltpu.sync_copy(zbuf, acc.at[pl.ds(s * RC, RZ)])
    pltpu.sync_copy(zbuf, acc.at[pl.ds(s * RC + RZ, RZ)])
    plsc.subcore_barrier()

    def group(g, carry):
        for b in range(KDEG):
            j = g * KDEG + b

            @pl.when(j >= KDEG)
            def _():
                pltpu.make_async_copy(ones_v, acc.at[dst_t.at[0]],
                                      sems[b]).wait()

            pltpu.make_async_copy(ones_v, acc.at[dst_t.at[j]],
                                  sems[b]).start(add=True)
        return carry

    lax.fori_loop(0, STEPS // KDEG, group, 0)
    for b in range(KDEG):
        pltpu.make_async_copy(ones_v, acc.at[dst_t.at[0]], sems[b]).wait()
    plsc.subcore_barrier()
    pltpu.sync_copy(acc.at[pl.ds(s * RC, RC)], out_hbm.at[c, pl.ds(s * RC, RC)])


_sc_deg = pl.kernel(
    _sc_deg_body,
    out_type=jax.ShapeDtypeStruct((NC, NPAD, 16), jnp.float32),
    mesh=_MESH,
    compiler_params=pltpu.CompilerParams(use_tc_tiling_on_sc=False),
    scratch_types=[
        pltpu.VMEM((STEPS, B), jnp.int32),
        pltpu.VMEM((B, 16), jnp.float32),
        pltpu.VMEM((RZ, 16), jnp.float32),
        pltpu.VMEM_SHARED((NPAD, 16), jnp.float32),
    ] + [pltpu.SemaphoreType.DMA] * KDEG,
)


def _sc_agg_body(ulo_hbm, uhi_hbm, er_hbm, out_hbm, src_t, dst_t,
                 r0, r1, r2, r3, zbuf, acc, *sems):
    c = lax.axis_index("c")
    s = lax.axis_index("s")
    wid = s * NC + c
    rows = (r0, r1, r2, r3)
    gsem = sems[:NBUF]
    ssem = sems[NBUF:]

    pltpu.sync_copy(er_hbm.at[0, wid], src_t)
    pltpu.sync_copy(er_hbm.at[1, wid], dst_t)
    _zero_fill(zbuf, RZ, H)

    for half, u_hbm in ((0, ulo_hbm), (1, uhi_hbm)):
        for b in range(2):
            pltpu.make_async_copy(u_hbm.at[src_t.at[b]], rows[b],
                                  gsem[b]).start()

        pltpu.sync_copy(zbuf, acc.at[pl.ds(s * RC, RZ)])
        pltpu.sync_copy(zbuf, acc.at[pl.ds(s * RC + RZ, RZ)])
        plsc.subcore_barrier()

        def group(g, carry):
            for b in range(NBUF):
                j = g * NBUF + b
                pltpu.make_async_copy(u_hbm.at[src_t.at[0]], rows[b],
                                      gsem[b]).wait()
                pltpu.make_async_copy(rows[b], acc.at[dst_t.at[j]],
                                      ssem[b]).start(add=True)
                nj = j + 2
                b2 = (b + 2) % NBUF

                @pl.when(nj < STEPS)
                def _():
                    @pl.when(j >= 2)
                    def _():
                        pltpu.make_async_copy(rows[b2], acc.at[dst_t.at[0]],
                                              ssem[b2]).wait()

                    pltpu.make_async_copy(u_hbm.at[src_t.at[nj]], rows[b2],
                                          gsem[b2]).start()
            return carry

        lax.fori_loop(0, STEPS // NBUF, group, 0)
        for b in range(NBUF):
            pltpu.make_async_copy(rows[b], acc.at[dst_t.at[0]],
                                  ssem[b]).wait()
        plsc.subcore_barrier()
        pltpu.sync_copy(acc.at[pl.ds(s * RC, RC)],
                        out_hbm.at[c, half, pl.ds(s * RC, RC)])
        if half == 0:
            plsc.subcore_barrier()


_sc_agg = pl.kernel(
    _sc_agg_body,
    out_type=jax.ShapeDtypeStruct((NC, 2, NPAD, H), jnp.float32),
    mesh=_MESH,
    compiler_params=pltpu.CompilerParams(use_tc_tiling_on_sc=False),
    scratch_types=[
        pltpu.VMEM((STEPS, B), jnp.int32),
        pltpu.VMEM((STEPS, B), jnp.int32),
    ] + [pltpu.VMEM((B, H), jnp.float32)] * NBUF + [
        pltpu.VMEM((RZ, H), jnp.float32),
        pltpu.VMEM_SHARED((NPAD, H), jnp.float32),
    ] + [pltpu.SemaphoreType.DMA] * (2 * NBUF),
)


def _tc_a_body(degp_ref, x_ref, ulo_ref, uhi_ref, dinv_ref):
    deg = (degp_ref[0, :N, 0:1] + degp_ref[1, :N, 0:1]
           + 1.0)
    dinv = lax.rsqrt(deg)
    dinv_ref[...] = dinv
    u = x_ref[...] * dinv
    ulo_ref[...] = u[:, :H]
    uhi_ref[...] = u[:, H:]


_tc_a = pl.pallas_call(
    _tc_a_body,
    out_shape=(
        jax.ShapeDtypeStruct((N, H), jnp.float32),
        jax.ShapeDtypeStruct((N, H), jnp.float32),
        jax.ShapeDtypeStruct((N, 1), jnp.float32),
    ),
)


def _assemble(p_ref, ulo_ref, uhi_ref, dinv):
    lo = p_ref[0, 0, :N, :] + p_ref[1, 0, :N, :] + ulo_ref[...]
    hi = p_ref[0, 1, :N, :] + p_ref[1, 1, :N, :] + uhi_ref[...]
    return jnp.concatenate([lo, hi], axis=1) * dinv


def _tc_b_body(p_ref, ulo_ref, uhi_ref, dinv_ref, W1_ref, g_ref, b_ref,
               u2lo_ref, u2hi_ref):
    dinv = dinv_ref[...]
    agg = _assemble(p_ref, ulo_ref, uhi_ref, dinv)
    hp = jnp.dot(agg, W1_ref[...], preferred_element_type=jnp.float32)
    m = jnp.mean(hp, axis=0, keepdims=True)
    v = jnp.mean((hp - m) ** 2, axis=0, keepdims=True)
    h = (hp - m) * lax.rsqrt(v + EPS) * g_ref[...] + b_ref[...]
    h = jnp.maximum(h, 0.0)
    u2 = h * dinv
    u2lo_ref[...] = u2[:, :H]
    u2hi_ref[...] = u2[:, H:]


_tc_b = pl.pallas_call(
    _tc_b_body,
    out_shape=(
        jax.ShapeDtypeStruct((N, H), jnp.float32),
        jax.ShapeDtypeStruct((N, H), jnp.float32),
    ),
)


def _tc_c_body(p_ref, u2lo_ref, u2hi_ref, dinv_ref, Wmu_ref, bmu_ref,
               Wls_ref, bls_ref, mu_ref, ls_ref):
    agg = _assemble(p_ref, u2lo_ref, u2hi_ref, dinv_ref[...])
    mu_ref[...] = jnp.dot(agg, Wmu_ref[...],
                          preferred_element_type=jnp.float32) + bmu_ref[...]
    ls_ref[...] = jnp.dot(agg, Wls_ref[...],
                          preferred_element_type=jnp.float32) + bls_ref[...]


_tc_c = pl.pallas_call(
    _tc_c_body,
    out_shape=(
        jax.ShapeDtypeStruct((N, D), jnp.float32),
        jax.ShapeDtypeStruct((N, D), jnp.float32),
    ),
)


@jax.jit
def kernel(x, edge_index, W1, b1, gamma, beta, Wmu, bmu, Wls, bls):
    er = edge_index.reshape(2, NW, STEPS, B)

    degp = _sc_deg(er)
    ulo, uhi, dinv = _tc_a(degp, x)

    p1 = _sc_agg(ulo, uhi, er)
    u2lo, u2hi = _tc_b(p1, ulo, uhi, dinv, W1, gamma, beta)

    p2 = _sc_agg(u2lo, u2hi, er)
    mu, ls = _tc_c(p2, u2lo, u2hi, dinv, Wmu, bmu, Wls, bls)
    return (mu, ls)


# speedup vs baseline: 32.0164x; 1.0008x over previous
import jax
import jax.numpy as jnp
from jax import lax
from jax.experimental import pallas as pl
from jax.experimental.pallas import tpu as pltpu
from jax.experimental.pallas import tpu_sc as plsc

N = 10000
E = 320000
D = 128
H = D // 2
EPS = 1e-5

NC = 2
NS = 16
NW = NC * NS
EPT = E // NW
B = 125
STEPS = EPT // B
NBUF = 5
KDEG = 10
NPAD = 10240
RZ = NPAD // NW
RC = NPAD // NS

_MESH = plsc.VectorSubcoreMesh(core_axis_name="c", subcore_axis_name="s")


def _zero_fill(buf, rows, width):
    z = jnp.zeros((16,), jnp.float32)

    def row(i, carry):
        for j in range(width // 16):
            buf[i, pl.ds(j * 16, 16)] = z
        return carry

    lax.fori_loop(0, rows, row, 0)


def _sc_deg_body(er_hbm, out_hbm, dst_t, ones_v, zbuf, acc, *sems):
    c = lax.axis_index("c")
    s = lax.axis_index("s")
    wid = s * NC + c

    one = jnp.ones((16,), jnp.float32)

    def orow(i, carry):
        ones_v[i, pl.ds(0, 16)] = one
        return carry

    lax.fori_loop(0, B, orow, 0)
    pltpu.sync_copy(er_hbm.at[1, wid], dst_t)
    _zero_fill(zbuf, RZ, 16)
    pltpu.sync_copy(zbuf, acc.at[pl.ds(s * RC, RZ)])
    pltpu.sync_copy(zbuf, acc.at[pl.ds(s * RC + RZ, RZ)])
    plsc.subcore_barrier()

    def group(g, carry):
        for b in range(KDEG):
            j = g * KDEG + b

            @pl.when(j >= KDEG)
            def _():
                pltpu.make_async_copy(ones_v, acc.at[dst_t.at[0]],
                                      sems[b]).wait()

            pltpu.make_async_copy(ones_v, acc.at[dst_t.at[j]],
                                  sems[b]).start(add=True)
        return carry

    lax.fori_loop(0, STEPS // KDEG, group, 0)
    for b in range(KDEG):
        pltpu.make_async_copy(ones_v, acc.at[dst_t.at[0]], sems[b]).wait()
    plsc.subcore_barrier()
    pltpu.sync_copy(acc.at[pl.ds(s * RC, RC)], out_hbm.at[c, pl.ds(s * RC, RC)])


_sc_deg = pl.kernel(
    _sc_deg_body,
    out_type=jax.ShapeDtypeStruct((NC, NPAD, 16), jnp.float32),
    mesh=_MESH,
    compiler_params=pltpu.CompilerParams(use_tc_tiling_on_sc=False),
    scratch_types=[
        pltpu.VMEM((STEPS, B), jnp.int32),
        pltpu.VMEM((B, 16), jnp.float32),
        pltpu.VMEM((RZ, 16), jnp.float32),
        pltpu.VMEM_SHARED((NPAD, 16), jnp.float32),
    ] + [pltpu.SemaphoreType.DMA] * KDEG,
)


def _sc_agg_body(ulo_hbm, uhi_hbm, er_hbm, out_hbm, src_t, dst_t,
                 r0, r1, r2, r3, r4, zbuf, acc, *sems):
    c = lax.axis_index("c")
    s = lax.axis_index("s")
    wid = s * NC + c
    rows = (r0, r1, r2, r3, r4)
    gsem = sems[:NBUF]
    ssem = sems[NBUF:]

    pltpu.sync_copy(er_hbm.at[0, wid], src_t)
    pltpu.sync_copy(er_hbm.at[1, wid], dst_t)
    _zero_fill(zbuf, RZ, H)

    for half, u_hbm in ((0, ulo_hbm), (1, uhi_hbm)):
        for b in range(2):
            pltpu.make_async_copy(u_hbm.at[src_t.at[b]], rows[b],
                                  gsem[b]).start()

        pltpu.sync_copy(zbuf, acc.at[pl.ds(s * RC, RZ)])
        pltpu.sync_copy(zbuf, acc.at[pl.ds(s * RC + RZ, RZ)])
        plsc.subcore_barrier()

        def group(g, carry):
            for b in range(NBUF):
                j = g * NBUF + b
                pltpu.make_async_copy(u_hbm.at[src_t.at[0]], rows[b],
                                      gsem[b]).wait()
                pltpu.make_async_copy(rows[b], acc.at[dst_t.at[j]],
                                      ssem[b]).start(add=True)
                nj = j + 2
                b2 = (b + 2) % NBUF

                @pl.when(nj < STEPS)
                def _():
                    @pl.when(j >= 3)
                    def _():
                        pltpu.make_async_copy(rows[b2], acc.at[dst_t.at[0]],
                                              ssem[b2]).wait()

                    pltpu.make_async_copy(u_hbm.at[src_t.at[nj]], rows[b2],
                                          gsem[b2]).start()
            return carry

        lax.fori_loop(0, STEPS // NBUF, group, 0)
        for b in range(NBUF):
            pltpu.make_async_copy(rows[b], acc.at[dst_t.at[0]],
                                  ssem[b]).wait()
        plsc.subcore_barrier()
        pltpu.sync_copy(acc.at[pl.ds(s * RC, RC)],
                        out_hbm.at[c, half, pl.ds(s * RC, RC)])
        if half == 0:
            plsc.subcore_barrier()


_sc_agg = pl.kernel(
    _sc_agg_body,
    out_type=jax.ShapeDtypeStruct((NC, 2, NPAD, H), jnp.float32),
    mesh=_MESH,
    compiler_params=pltpu.CompilerParams(use_tc_tiling_on_sc=False),
    scratch_types=[
        pltpu.VMEM((STEPS, B), jnp.int32),
        pltpu.VMEM((STEPS, B), jnp.int32),
    ] + [pltpu.VMEM((B, H), jnp.float32)] * NBUF + [
        pltpu.VMEM((RZ, H), jnp.float32),
        pltpu.VMEM_SHARED((NPAD, H), jnp.float32),
    ] + [pltpu.SemaphoreType.DMA] * (2 * NBUF),
)


def _tc_a_body(degp_ref, x_ref, ulo_ref, uhi_ref, dinv_ref):
    deg = (degp_ref[0, :N, 0:1] + degp_ref[1, :N, 0:1]
           + 1.0)
    dinv = lax.rsqrt(deg)
    dinv_ref[...] = dinv
    u = x_ref[...] * dinv
    ulo_ref[...] = u[:, :H]
    uhi_ref[...] = u[:, H:]


_tc_a = pl.pallas_call(
    _tc_a_body,
    out_shape=(
        jax.ShapeDtypeStruct((N, H), jnp.float32),
        jax.ShapeDtypeStruct((N, H), jnp.float32),
        jax.ShapeDtypeStruct((N, 1), jnp.float32),
    ),
)


def _assemble(p_ref, ulo_ref, uhi_ref, dinv):
    lo = p_ref[0, 0, :N, :] + p_ref[1, 0, :N, :] + ulo_ref[...]
    hi = p_ref[0, 1, :N, :] + p_ref[1, 1, :N, :] + uhi_ref[...]
    return jnp.concatenate([lo, hi], axis=1) * dinv


def _tc_b_body(p_ref, ulo_ref, uhi_ref, dinv_ref, W1_ref, g_ref, b_ref,
               u2lo_ref, u2hi_ref):
    dinv = dinv_ref[...]
    agg = _assemble(p_ref, ulo_ref, uhi_ref, dinv)
    hp = jnp.dot(agg, W1_ref[...], preferred_element_type=jnp.float32)
    m = jnp.mean(hp, axis=0, keepdims=True)
    v = jnp.mean((hp - m) ** 2, axis=0, keepdims=True)
    h = (hp - m) * lax.rsqrt(v + EPS) * g_ref[...] + b_ref[...]
    h = jnp.maximum(h, 0.0)
    u2 = h * dinv
    u2lo_ref[...] = u2[:, :H]
    u2hi_ref[...] = u2[:, H:]


_tc_b = pl.pallas_call(
    _tc_b_body,
    out_shape=(
        jax.ShapeDtypeStruct((N, H), jnp.float32),
        jax.ShapeDtypeStruct((N, H), jnp.float32),
    ),
)


def _tc_c_body(p_ref, u2lo_ref, u2hi_ref, dinv_ref, Wmu_ref, bmu_ref,
               Wls_ref, bls_ref, mu_ref, ls_ref):
    agg = _assemble(p_ref, u2lo_ref, u2hi_ref, dinv_ref[...])
    mu_ref[...] = jnp.dot(agg, Wmu_ref[...],
                          preferred_element_type=jnp.float32) + bmu_ref[...]
    ls_ref[...] = jnp.dot(agg, Wls_ref[...],
                          preferred_element_type=jnp.float32) + bls_ref[...]


_tc_c = pl.pallas_call(
    _tc_c_body,
    out_shape=(
        jax.ShapeDtypeStruct((N, D), jnp.float32),
        jax.ShapeDtypeStruct((N, D), jnp.float32),
    ),
)


@jax.jit
def kernel(x, edge_index, W1, b1, gamma, beta, Wmu, bmu, Wls, bls):
    er = edge_index.reshape(2, NW, STEPS, B)

    degp = _sc_deg(er)
    ulo, uhi, dinv = _tc_a(degp, x)

    p1 = _sc_agg(ulo, uhi, er)
    u2lo, u2hi = _tc_b(p1, ulo, uhi, dinv, W1, gamma, beta)

    p2 = _sc_agg(u2lo, u2hi, er)
    mu, ls = _tc_c(p2, u2lo, u2hi, dinv, Wmu, bmu, Wls, bls)
    return (mu, ls)


# speedup vs baseline: 32.0522x; 1.0011x over previous
import jax
import jax.numpy as jnp
from jax import lax
from jax.experimental import pallas as pl
from jax.experimental.pallas import tpu as pltpu
from jax.experimental.pallas import tpu_sc as plsc

N = 10000
E = 320000
D = 128
H = D // 2
EPS = 1e-5

NC = 2
NS = 16
NW = NC * NS
EPT = E // NW
B = 125
STEPS = EPT // B
NBUF = 5
KDEG = 10
NPAD = 10240
RZ = NPAD // NW
RC = NPAD // NS

_MESH = plsc.VectorSubcoreMesh(core_axis_name="c", subcore_axis_name="s")


def _zero_fill(buf, rows, width):
    z = jnp.zeros((16,), jnp.float32)

    def row(i, carry):
        for j in range(width // 16):
            buf[i, pl.ds(j * 16, 16)] = z
        return carry

    lax.fori_loop(0, rows, row, 0)


def _sc_deg_body(er_hbm, out_hbm, dst_t, ones_v, zbuf, acc, *sems):
    c = lax.axis_index("c")
    s = lax.axis_index("s")
    wid = s * NC + c

    one = jnp.ones((16,), jnp.float32)

    def orow(i, carry):
        ones_v[i, pl.ds(0, 16)] = one
        return carry

    lax.fori_loop(0, B, orow, 0)
    pltpu.sync_copy(er_hbm.at[1, wid], dst_t)
    _zero_fill(zbuf, RZ, 16)
    pltpu.sync_copy(zbuf, acc.at[pl.ds(s * RC, RZ)])
    pltpu.sync_copy(zbuf, acc.at[pl.ds(s * RC + RZ, RZ)])
    plsc.subcore_barrier()

    def group(g, carry):
        for b in range(KDEG):
            j = g * KDEG + b

            @pl.when(j >= KDEG)
            def _():
                pltpu.make_async_copy(ones_v, acc.at[dst_t.at[0]],
                                      sems[b]).wait()

            pltpu.make_async_copy(ones_v, acc.at[dst_t.at[j]],
                                  sems[b]).start(add=True)
        return carry

    lax.fori_loop(0, STEPS // KDEG, group, 0)
    for b in range(KDEG):
        pltpu.make_async_copy(ones_v, acc.at[dst_t.at[0]], sems[b]).wait()
    plsc.subcore_barrier()
    pltpu.sync_copy(acc.at[pl.ds(s * RC, RC)], out_hbm.at[c, pl.ds(s * RC, RC)])


_sc_deg = pl.kernel(
    _sc_deg_body,
    out_type=jax.ShapeDtypeStruct((NC, NPAD, 16), jnp.float32),
    mesh=_MESH,
    compiler_params=pltpu.CompilerParams(use_tc_tiling_on_sc=False, skip_device_barrier=True),
    scratch_types=[
        pltpu.VMEM((STEPS, B), jnp.int32),
        pltpu.VMEM((B, 16), jnp.float32),
        pltpu.VMEM((RZ, 16), jnp.float32),
        pltpu.VMEM_SHARED((NPAD, 16), jnp.float32),
    ] + [pltpu.SemaphoreType.DMA] * KDEG,
)


def _sc_agg_body(ulo_hbm, uhi_hbm, er_hbm, out_hbm, src_t, dst_t,
                 r0, r1, r2, r3, r4, zbuf, acc, *sems):
    c = lax.axis_index("c")
    s = lax.axis_index("s")
    wid = s * NC + c
    rows = (r0, r1, r2, r3, r4)
    gsem = sems[:NBUF]
    ssem = sems[NBUF:]

    pltpu.sync_copy(er_hbm.at[0, wid], src_t)
    pltpu.sync_copy(er_hbm.at[1, wid], dst_t)
    _zero_fill(zbuf, RZ, H)

    for half, u_hbm in ((0, ulo_hbm), (1, uhi_hbm)):
        for b in range(2):
            pltpu.make_async_copy(u_hbm.at[src_t.at[b]], rows[b],
                                  gsem[b]).start()

        pltpu.sync_copy(zbuf, acc.at[pl.ds(s * RC, RZ)])
        pltpu.sync_copy(zbuf, acc.at[pl.ds(s * RC + RZ, RZ)])
        plsc.subcore_barrier()

        def group(g, carry):
            for b in range(NBUF):
                j = g * NBUF + b
                pltpu.make_async_copy(u_hbm.at[src_t.at[0]], rows[b],
                                      gsem[b]).wait()
                pltpu.make_async_copy(rows[b], acc.at[dst_t.at[j]],
                                      ssem[b]).start(add=True)
                nj = j + 2
                b2 = (b + 2) % NBUF

                @pl.when(nj < STEPS)
                def _():
                    @pl.when(j >= 3)
                    def _():
                        pltpu.make_async_copy(rows[b2], acc.at[dst_t.at[0]],
                                              ssem[b2]).wait()

                    pltpu.make_async_copy(u_hbm.at[src_t.at[nj]], rows[b2],
                                          gsem[b2]).start()
            return carry

        lax.fori_loop(0, STEPS // NBUF, group, 0)
        for b in range(NBUF):
            pltpu.make_async_copy(rows[b], acc.at[dst_t.at[0]],
                                  ssem[b]).wait()
        plsc.subcore_barrier()
        pltpu.sync_copy(acc.at[pl.ds(s * RC, RC)],
                        out_hbm.at[c, half, pl.ds(s * RC, RC)])
        if half == 0:
            plsc.subcore_barrier()


_sc_agg = pl.kernel(
    _sc_agg_body,
    out_type=jax.ShapeDtypeStruct((NC, 2, NPAD, H), jnp.float32),
    mesh=_MESH,
    compiler_params=pltpu.CompilerParams(use_tc_tiling_on_sc=False, skip_device_barrier=True),
    scratch_types=[
        pltpu.VMEM((STEPS, B), jnp.int32),
        pltpu.VMEM((STEPS, B), jnp.int32),
    ] + [pltpu.VMEM((B, H), jnp.float32)] * NBUF + [
        pltpu.VMEM((RZ, H), jnp.float32),
        pltpu.VMEM_SHARED((NPAD, H), jnp.float32),
    ] + [pltpu.SemaphoreType.DMA] * (2 * NBUF),
)


def _tc_a_body(degp_ref, x_ref, ulo_ref, uhi_ref, dinv_ref):
    deg = (degp_ref[0, :N, 0:1] + degp_ref[1, :N, 0:1]
           + 1.0)
    dinv = lax.rsqrt(deg)
    dinv_ref[...] = dinv
    u = x_ref[...] * dinv
    ulo_ref[...] = u[:, :H]
    uhi_ref[...] = u[:, H:]


_tc_a = pl.pallas_call(
    _tc_a_body,
    out_shape=(
        jax.ShapeDtypeStruct((N, H), jnp.float32),
        jax.ShapeDtypeStruct((N, H), jnp.float32),
        jax.ShapeDtypeStruct((N, 1), jnp.float32),
    ),
)


def _assemble(p_ref, ulo_ref, uhi_ref, dinv):
    lo = p_ref[0, 0, :N, :] + p_ref[1, 0, :N, :] + ulo_ref[...]
    hi = p_ref[0, 1, :N, :] + p_ref[1, 1, :N, :] + uhi_ref[...]
    return jnp.concatenate([lo, hi], axis=1) * dinv


def _tc_b_body(p_ref, ulo_ref, uhi_ref, dinv_ref, W1_ref, g_ref, b_ref,
               u2lo_ref, u2hi_ref):
    dinv = dinv_ref[...]
    agg = _assemble(p_ref, ulo_ref, uhi_ref, dinv)
    hp = jnp.dot(agg, W1_ref[...], preferred_element_type=jnp.float32)
    m = jnp.mean(hp, axis=0, keepdims=True)
    v = jnp.mean((hp - m) ** 2, axis=0, keepdims=True)
    h = (hp - m) * lax.rsqrt(v + EPS) * g_ref[...] + b_ref[...]
    h = jnp.maximum(h, 0.0)
    u2 = h * dinv
    u2lo_ref[...] = u2[:, :H]
    u2hi_ref[...] = u2[:, H:]


_tc_b = pl.pallas_call(
    _tc_b_body,
    out_shape=(
        jax.ShapeDtypeStruct((N, H), jnp.float32),
        jax.ShapeDtypeStruct((N, H), jnp.float32),
    ),
)


def _tc_c_body(p_ref, u2lo_ref, u2hi_ref, dinv_ref, Wmu_ref, bmu_ref,
               Wls_ref, bls_ref, mu_ref, ls_ref):
    agg = _assemble(p_ref, u2lo_ref, u2hi_ref, dinv_ref[...])
    mu_ref[...] = jnp.dot(agg, Wmu_ref[...],
                          preferred_element_type=jnp.float32) + bmu_ref[...]
    ls_ref[...] = jnp.dot(agg, Wls_ref[...],
                          preferred_element_type=jnp.float32) + bls_ref[...]


_tc_c = pl.pallas_call(
    _tc_c_body,
    out_shape=(
        jax.ShapeDtypeStruct((N, D), jnp.float32),
        jax.ShapeDtypeStruct((N, D), jnp.float32),
    ),
)


@jax.jit
def kernel(x, edge_index, W1, b1, gamma, beta, Wmu, bmu, Wls, bls):
    er = edge_index.reshape(2, NW, STEPS, B)

    degp = _sc_deg(er)
    ulo, uhi, dinv = _tc_a(degp, x)

    p1 = _sc_agg(ulo, uhi, er)
    u2lo, u2hi = _tc_b(p1, ulo, uhi, dinv, W1, gamma, beta)

    p2 = _sc_agg(u2lo, u2hi, er)
    mu, ls = _tc_c(p2, u2lo, u2hi, dinv, Wmu, bmu, Wls, bls)
    return (mu, ls)


# speedup vs baseline: 32.0624x; 1.0003x over previous
import jax
import jax.numpy as jnp
from jax import lax
from jax.experimental import pallas as pl
from jax.experimental.pallas import tpu as pltpu
from jax.experimental.pallas import tpu_sc as plsc

N = 10000
E = 320000
D = 128
H = D // 2
EPS = 1e-5

NC = 2
NS = 16
NW = NC * NS
EPT = E // NW
B = 125
STEPS = EPT // B
NBUF = 5
KDEG = 10
NPAD = 10240
RZ = NPAD // NW
RC = NPAD // NS

_MESH = plsc.VectorSubcoreMesh(core_axis_name="c", subcore_axis_name="s")


def _zero_fill(buf, rows, width):
    z = jnp.zeros((16,), jnp.float32)

    def row(i, carry):
        for j in range(width // 16):
            buf[i, pl.ds(j * 16, 16)] = z
        return carry

    lax.fori_loop(0, rows, row, 0)


def _sc_deg_body(er_hbm, out_hbm, dst_t, ones_v, zbuf, acc, *sems):
    c = lax.axis_index("c")
    s = lax.axis_index("s")
    wid = s * NC + c

    one = jnp.ones((16,), jnp.float32)

    def orow(i, carry):
        ones_v[i, pl.ds(0, 16)] = one
        return carry

    lax.fori_loop(0, B, orow, 0)
    pltpu.sync_copy(er_hbm.at[1, wid], dst_t)
    _zero_fill(zbuf, RZ, 16)
    pltpu.sync_copy(zbuf, acc.at[pl.ds(s * RC, RZ)])
    pltpu.sync_copy(zbuf, acc.at[pl.ds(s * RC + RZ, RZ)])
    plsc.subcore_barrier()

    def group(g, carry):
        for b in range(KDEG):
            j = g * KDEG + b

            @pl.when(j >= KDEG)
            def _():
                pltpu.make_async_copy(ones_v, acc.at[dst_t.at[0]],
                                      sems[b]).wait()

            pltpu.make_async_copy(ones_v, acc.at[dst_t.at[j]],
                                  sems[b]).start(add=True)
        return carry

    lax.fori_loop(0, STEPS // KDEG, group, 0)
    for b in range(KDEG):
        pltpu.make_async_copy(ones_v, acc.at[dst_t.at[0]], sems[b]).wait()
    plsc.subcore_barrier()
    pltpu.sync_copy(acc.at[pl.ds(s * RC, RC)], out_hbm.at[c, pl.ds(s * RC, RC)])


_sc_deg = pl.kernel(
    _sc_deg_body,
    out_type=jax.ShapeDtypeStruct((NC, NPAD, 16), jnp.float32),
    mesh=_MESH,
    compiler_params=pltpu.CompilerParams(use_tc_tiling_on_sc=False),
    scratch_types=[
        pltpu.VMEM((STEPS, B), jnp.int32),
        pltpu.VMEM((B, 16), jnp.float32),
        pltpu.VMEM((RZ, 16), jnp.float32),
        pltpu.VMEM_SHARED((NPAD, 16), jnp.float32),
    ] + [pltpu.SemaphoreType.DMA] * KDEG,
)


def _sc_agg_body(ulo_hbm, uhi_hbm, er_hbm, out_hbm, src_t, dst_t,
                 r0, r1, r2, r3, r4, zbuf, acc, *sems):
    c = lax.axis_index("c")
    s = lax.axis_index("s")
    wid = s * NC + c
    rows = (r0, r1, r2, r3, r4)
    gsem = sems[:NBUF]
    ssem = sems[NBUF:]

    pltpu.sync_copy(er_hbm.at[0, wid], src_t)
    pltpu.sync_copy(er_hbm.at[1, wid], dst_t)
    _zero_fill(zbuf, RZ, H)

    for half, u_hbm in ((0, ulo_hbm), (1, uhi_hbm)):
        for b in range(2):
            pltpu.make_async_copy(u_hbm.at[src_t.at[b]], rows[b],
                                  gsem[b]).start()

        pltpu.sync_copy(zbuf, acc.at[pl.ds(s * RC, RZ)])
        pltpu.sync_copy(zbuf, acc.at[pl.ds(s * RC + RZ, RZ)])
        plsc.subcore_barrier()

        def group(g, carry):
            for b in range(NBUF):
                j = g * NBUF + b
                pltpu.make_async_copy(u_hbm.at[src_t.at[0]], rows[b],
                                      gsem[b]).wait()
                pltpu.make_async_copy(rows[b], acc.at[dst_t.at[j]],
                                      ssem[b]).start(add=True)
                nj = j + 2
                b2 = (b + 2) % NBUF

                @pl.when(nj < STEPS)
                def _():
                    @pl.when(j >= 3)
                    def _():
                        pltpu.make_async_copy(rows[b2], acc.at[dst_t.at[0]],
                                              ssem[b2]).wait()

                    pltpu.make_async_copy(u_hbm.at[src_t.at[nj]], rows[b2],
                                          gsem[b2]).start()
            return carry

        lax.fori_loop(0, STEPS // NBUF, group, 0)
        for b in range(NBUF):
            pltpu.make_async_copy(rows[b], acc.at[dst_t.at[0]],
                                  ssem[b]).wait()
        plsc.subcore_barrier()
        pltpu.sync_copy(acc.at[pl.ds(s * RC, RC)],
                        out_hbm.at[c, half, pl.ds(s * RC, RC)])
        if half == 0:
            plsc.subcore_barrier()


_sc_agg = pl.kernel(
    _sc_agg_body,
    out_type=jax.ShapeDtypeStruct((NC, 2, NPAD, H), jnp.float32),
    mesh=_MESH,
    compiler_params=pltpu.CompilerParams(use_tc_tiling_on_sc=False),
    scratch_types=[
        pltpu.VMEM((STEPS, B), jnp.int32),
        pltpu.VMEM((STEPS, B), jnp.int32),
    ] + [pltpu.VMEM((B, H), jnp.float32)] * NBUF + [
        pltpu.VMEM((RZ, H), jnp.float32),
        pltpu.VMEM_SHARED((NPAD, H), jnp.float32),
    ] + [pltpu.SemaphoreType.DMA] * (2 * NBUF),
)


def _tc_a_body(degp_ref, x_ref, ulo_ref, uhi_ref, dinv_ref):
    deg = (degp_ref[0, :N, 0:1] + degp_ref[1, :N, 0:1]
           + 1.0)
    dinv = lax.rsqrt(deg)
    dinv_ref[...] = dinv
    u = x_ref[...] * dinv
    ulo_ref[...] = u[:, :H]
    uhi_ref[...] = u[:, H:]


_tc_a = pl.pallas_call(
    _tc_a_body,
    out_shape=(
        jax.ShapeDtypeStruct((N, H), jnp.float32),
        jax.ShapeDtypeStruct((N, H), jnp.float32),
        jax.ShapeDtypeStruct((N, 1), jnp.float32),
    ),
)


def _assemble(p_ref, ulo_ref, uhi_ref, dinv):
    lo = p_ref[0, 0, :N, :] + p_ref[1, 0, :N, :] + ulo_ref[...]
    hi = p_ref[0, 1, :N, :] + p_ref[1, 1, :N, :] + uhi_ref[...]
    return jnp.concatenate([lo, hi], axis=1) * dinv


def _tc_b_body(p_ref, ulo_ref, uhi_ref, dinv_ref, W1_ref, g_ref, b_ref,
               u2lo_ref, u2hi_ref):
    dinv = dinv_ref[...]
    agg = _assemble(p_ref, ulo_ref, uhi_ref, dinv)
    hp = jnp.dot(agg, W1_ref[...], preferred_element_type=jnp.float32)
    m = jnp.mean(hp, axis=0, keepdims=True)
    v = jnp.mean((hp - m) ** 2, axis=0, keepdims=True)
    h = (hp - m) * lax.rsqrt(v + EPS) * g_ref[...] + b_ref[...]
    h = jnp.maximum(h, 0.0)
    u2 = h * dinv
    u2lo_ref[...] = u2[:, :H]
    u2hi_ref[...] = u2[:, H:]


_tc_b = pl.pallas_call(
    _tc_b_body,
    out_shape=(
        jax.ShapeDtypeStruct((N, H), jnp.float32),
        jax.ShapeDtypeStruct((N, H), jnp.float32),
    ),
)


def _tc_c_body(p_ref, u2lo_ref, u2hi_ref, dinv_ref, Wmu_ref, bmu_ref,
               Wls_ref, bls_ref, mu_ref, ls_ref):
    agg = _assemble(p_ref, u2lo_ref, u2hi_ref, dinv_ref[...])
    mu_ref[...] = jnp.dot(agg, Wmu_ref[...],
                          preferred_element_type=jnp.float32) + bmu_ref[...]
    ls_ref[...] = jnp.dot(agg, Wls_ref[...],
                          preferred_element_type=jnp.float32) + bls_ref[...]


_tc_c = pl.pallas_call(
    _tc_c_body,
    out_shape=(
        jax.ShapeDtypeStruct((N, D), jnp.float32),
        jax.ShapeDtypeStruct((N, D), jnp.float32),
    ),
)


@jax.jit
def kernel(x, edge_index, W1, b1, gamma, beta, Wmu, bmu, Wls, bls):
    er = edge_index.reshape(2, NW, STEPS, B)

    degp = _sc_deg(er)
    ulo, uhi, dinv = _tc_a(degp, x)

    p1 = _sc_agg(ulo, uhi, er)
    u2lo, u2hi = _tc_b(p1, ulo, uhi, dinv, W1, gamma, beta)

    p2 = _sc_agg(u2lo, u2hi, er)
    mu, ls = _tc_c(p2, u2lo, u2hi, dinv, Wmu, bmu, Wls, bls)
    return (mu, ls)
